# baseline (speedup 1.0000x reference)
import functools

import jax
import jax.numpy as jnp
from jax.experimental import pallas as pl

N_EXPERTS = 64
TOP_K = 2
BLOCK_TOKENS = 4096
IDX_MASK = N_EXPERTS - 1


def _router_block(x_ref, w_ref, probs_ref, idx_ref, wts_ref):
    x = x_ref[...]
    w = w_ref[...]
    logits = jax.lax.dot_general(
        x, w, (((1,), (1,)), ((), ())), preferred_element_type=jnp.float32
    )
    m = jnp.max(logits, axis=-1, keepdims=True)
    e = jnp.exp(logits - m)
    s = jnp.sum(e, axis=-1, keepdims=True)
    probs = e / s
    probs_ref[...] = probs

    iota = jax.lax.broadcasted_iota(jnp.int32, probs.shape, 1)
    keys = (probs.view(jnp.int32) & ~IDX_MASK) | (IDX_MASK - iota)
    k1 = jnp.max(keys, axis=-1, keepdims=True)
    k2 = jnp.max(jnp.where(keys == k1, jnp.int32(-1), keys), axis=-1, keepdims=True)
    k12 = jnp.concatenate([k1, k2], axis=-1)
    idx_ref[...] = IDX_MASK - (k12 & IDX_MASK)
    p12 = (k12 & ~IDX_MASK).view(jnp.float32)
    wts_ref[...] = p12 / (p12[:, 0:1] + p12[:, 1:2] + 1e-9)


@functools.partial(jax.jit, static_argnames=("interpret",))
def kernel(x, W, interpret=False):
    if x.ndim == 3:
        x = x.reshape(-1, x.shape[-1])
    n_tokens, d_model = x.shape
    n_blocks = n_tokens // BLOCK_TOKENS
    probs, idx, wts = pl.pallas_call(
        _router_block,
        grid=(n_blocks,),
        in_specs=[
            pl.BlockSpec((BLOCK_TOKENS, d_model), lambda i: (i, 0)),
            pl.BlockSpec((N_EXPERTS, d_model), lambda i: (0, 0)),
        ],
        out_specs=[
            pl.BlockSpec((BLOCK_TOKENS, N_EXPERTS), lambda i: (i, 0)),
            pl.BlockSpec((BLOCK_TOKENS, TOP_K), lambda i: (i, 0)),
            pl.BlockSpec((BLOCK_TOKENS, TOP_K), lambda i: (i, 0)),
        ],
        out_shape=[
            jax.ShapeDtypeStruct((n_tokens, N_EXPERTS), jnp.float32),
            jax.ShapeDtypeStruct((n_tokens, TOP_K), jnp.int32),
            jax.ShapeDtypeStruct((n_tokens, TOP_K), jnp.float32),
        ],
        interpret=interpret,
    )(x, W)
    return (probs, idx, wts)
